# baseline (device time: 1067202 ns/iter reference)
import jax
import jax.numpy as jnp
from jax import lax
from jax.experimental import pallas as pl
from jax.experimental.pallas import tpu as pltpu

M = 8192
N = 1024


def kernel(x):
    def body(x_ref, out_ref, stage_ref, local_sem, pack_sem, send_sem, recv_sem):
        my_x = lax.axis_index("x")
        my_y = lax.axis_index("y")
        my_z = lax.axis_index("z")
        peer_y = 1 - my_y

        barrier_sem = pltpu.get_barrier_semaphore()
        pl.semaphore_signal(
            barrier_sem, inc=1,
            device_id=(my_x, peer_y, my_z),
            device_id_type=pl.DeviceIdType.MESH,
        )
        pl.semaphore_wait(barrier_sem, 1)

        pack = pltpu.make_async_copy(
            x_ref.at[:, pl.ds(peer_y * N, N)],
            stage_ref,
            pack_sem,
        )
        pack.start()

        local = pltpu.make_async_copy(
            x_ref.at[:, pl.ds(my_y * N, N)],
            out_ref.at[pl.ds(my_y * M, M), :],
            local_sem,
        )
        local.start()

        pack.wait()

        rdma = pltpu.make_async_remote_copy(
            src_ref=stage_ref,
            dst_ref=out_ref.at[pl.ds(my_y * M, M), :],
            send_sem=send_sem,
            recv_sem=recv_sem,
            device_id=(my_x, peer_y, my_z),
            device_id_type=pl.DeviceIdType.MESH,
        )
        rdma.start()

        local.wait()
        rdma.wait()

    return pl.pallas_call(
        body,
        out_shape=jax.ShapeDtypeStruct((2 * M, N), jnp.float32),
        in_specs=[pl.BlockSpec(memory_space=pl.ANY)],
        out_specs=pl.BlockSpec(memory_space=pl.ANY),
        scratch_shapes=[
            pltpu.VMEM((M, N), jnp.float32),
            pltpu.SemaphoreType.DMA,
            pltpu.SemaphoreType.DMA,
            pltpu.SemaphoreType.DMA,
            pltpu.SemaphoreType.DMA,
        ],
        compiler_params=pltpu.CompilerParams(collective_id=0),
    )(x)


# device time: 419468 ns/iter; 2.5442x vs baseline; 2.5442x over previous
import jax
import jax.numpy as jnp
from jax import lax
from jax.experimental import pallas as pl
from jax.experimental.pallas import tpu as pltpu

M = 8192
N = 1024
CHUNK = 1024


def kernel(x):
    def body(x_ref, out_ref, stage_ref, loc_ref, in_sems, out_sems,
             pack_sem, send_sem, recv_sem):
        my_x = lax.axis_index("x")
        my_y = lax.axis_index("y")
        my_z = lax.axis_index("z")
        peer_y = 1 - my_y

        barrier_sem = pltpu.get_barrier_semaphore()
        pl.semaphore_signal(
            barrier_sem, inc=1,
            device_id=(my_x, peer_y, my_z),
            device_id_type=pl.DeviceIdType.MESH,
        )
        pl.semaphore_wait(barrier_sem, 1)

        pack = pltpu.make_async_copy(
            x_ref.at[:, pl.ds(peer_y * N, N)],
            stage_ref,
            pack_sem,
        )
        pack.start()
        pack.wait()

        rdma = pltpu.make_async_remote_copy(
            src_ref=stage_ref,
            dst_ref=out_ref.at[pl.ds(my_y * M, M), :],
            send_sem=send_sem,
            recv_sem=recv_sem,
            device_id=(my_x, peer_y, my_z),
            device_id_type=pl.DeviceIdType.MESH,
        )
        rdma.start()

        nch = M // CHUNK
        prev_out = [None, None]
        for c in range(nch):
            s = c % 2
            if prev_out[s] is not None:
                prev_out[s].wait()
            inc = pltpu.make_async_copy(
                x_ref.at[pl.ds(c * CHUNK, CHUNK), pl.ds(my_y * N, N)],
                loc_ref.at[s],
                in_sems.at[s],
            )
            inc.start()
            inc.wait()
            outc = pltpu.make_async_copy(
                loc_ref.at[s],
                out_ref.at[pl.ds(my_y * M + c * CHUNK, CHUNK), :],
                out_sems.at[s],
            )
            outc.start()
            prev_out[s] = outc
        for oc in prev_out:
            oc.wait()

        rdma.wait()

    return pl.pallas_call(
        body,
        out_shape=jax.ShapeDtypeStruct((2 * M, N), jnp.float32),
        in_specs=[pl.BlockSpec(memory_space=pl.ANY)],
        out_specs=pl.BlockSpec(memory_space=pl.ANY),
        scratch_shapes=[
            pltpu.VMEM((M, N), jnp.float32),
            pltpu.VMEM((2, CHUNK, N), jnp.float32),
            pltpu.SemaphoreType.DMA((2,)),
            pltpu.SemaphoreType.DMA((2,)),
            pltpu.SemaphoreType.DMA,
            pltpu.SemaphoreType.DMA,
            pltpu.SemaphoreType.DMA,
        ],
        compiler_params=pltpu.CompilerParams(
            collective_id=0,
            vmem_limit_bytes=64 * 1024 * 1024,
        ),
    )(x)


# device time: 410535 ns/iter; 2.5995x vs baseline; 1.0218x over previous
import jax
import jax.numpy as jnp
from jax import lax
from jax.experimental import pallas as pl
from jax.experimental.pallas import tpu as pltpu

M = 8192
N = 1024
CHUNK = 1024
NCH_R = 8
RROWS = M // NCH_R


def kernel(x):
    def body(x_ref, out_ref, stage_ref, loc_ref, in_sems, out_sems,
             pack_sems, send_sems, recv_sems):
        my_x = lax.axis_index("x")
        my_y = lax.axis_index("y")
        my_z = lax.axis_index("z")
        peer_y = 1 - my_y

        barrier_sem = pltpu.get_barrier_semaphore()
        pl.semaphore_signal(
            barrier_sem, inc=1,
            device_id=(my_x, peer_y, my_z),
            device_id_type=pl.DeviceIdType.MESH,
        )
        pl.semaphore_wait(barrier_sem, 1)

        packs = []
        for k in range(NCH_R):
            pk = pltpu.make_async_copy(
                x_ref.at[pl.ds(k * RROWS, RROWS), pl.ds(peer_y * N, N)],
                stage_ref.at[pl.ds(k * RROWS, RROWS), :],
                pack_sems.at[k],
            )
            pk.start()
            packs.append(pk)

        rdmas = []
        for k in range(NCH_R):
            packs[k].wait()
            op = pltpu.make_async_remote_copy(
                src_ref=stage_ref.at[pl.ds(k * RROWS, RROWS), :],
                dst_ref=out_ref.at[pl.ds(my_y * M + k * RROWS, RROWS), :],
                send_sem=send_sems.at[k],
                recv_sem=recv_sems.at[k],
                device_id=(my_x, peer_y, my_z),
                device_id_type=pl.DeviceIdType.MESH,
            )
            op.start()
            rdmas.append(op)

        nch = M // CHUNK
        prev_out = [None, None]
        for c in range(nch):
            s = c % 2
            if prev_out[s] is not None:
                prev_out[s].wait()
            inc = pltpu.make_async_copy(
                x_ref.at[pl.ds(c * CHUNK, CHUNK), pl.ds(my_y * N, N)],
                loc_ref.at[s],
                in_sems.at[s],
            )
            inc.start()
            inc.wait()
            outc = pltpu.make_async_copy(
                loc_ref.at[s],
                out_ref.at[pl.ds(my_y * M + c * CHUNK, CHUNK), :],
                out_sems.at[s],
            )
            outc.start()
            prev_out[s] = outc
        for oc in prev_out:
            oc.wait()

        for op in rdmas:
            op.wait()

    return pl.pallas_call(
        body,
        out_shape=jax.ShapeDtypeStruct((2 * M, N), jnp.float32),
        in_specs=[pl.BlockSpec(memory_space=pl.ANY)],
        out_specs=pl.BlockSpec(memory_space=pl.ANY),
        scratch_shapes=[
            pltpu.VMEM((M, N), jnp.float32),
            pltpu.VMEM((2, CHUNK, N), jnp.float32),
            pltpu.SemaphoreType.DMA((2,)),
            pltpu.SemaphoreType.DMA((2,)),
            pltpu.SemaphoreType.DMA((NCH_R,)),
            pltpu.SemaphoreType.DMA((NCH_R,)),
            pltpu.SemaphoreType.DMA((NCH_R,)),
        ],
        compiler_params=pltpu.CompilerParams(
            collective_id=0,
            vmem_limit_bytes=64 * 1024 * 1024,
        ),
    )(x)
